# Initial kernel scaffold; baseline (speedup 1.0000x reference)
import jax
import jax.numpy as jnp
from jax.experimental import pallas as pl


def kernel(x, edge_index, W1, b1, W2, b2):
    raise NotImplementedError("write your pallas kernel here")


# speedup vs baseline: 13.0886x; 13.0886x over previous
import dataclasses
import functools

import jax
import jax.numpy as jnp
from jax import lax
from jax.experimental import pallas as pl
from jax.experimental.pallas import tpu as pltpu
from jax.experimental.pallas import tpu_sc as plsc

N = 10000
E = 320000
D_IN = 128
D_HID = 256
D_OUT = 128
DH = 128

NC = 2
NS = 16
NW = NC * NS

CH = 128
E_ROWS = E // CH
RB = 80
NRB = N // RB

BR = 1000
f32 = jnp.float32


def _mesh():
    return plsc.VectorSubcoreMesh(core_axis_name="c", subcore_axis_name="s")


def _sc_params():
    cp = pltpu.CompilerParams()
    if "needs_layout_passes" in pltpu.CompilerParams.__dataclass_fields__:
        cp = dataclasses.replace(cp, needs_layout_passes=False)
    return cp


def _sc_hist(dst2):
    @functools.partial(
        pl.kernel,
        out_type=jax.ShapeDtypeStruct((NW, N), f32),
        mesh=_mesh(),
        scratch_types=[
            pltpu.VMEM((N,), f32),
            pltpu.VMEM((CH,), jnp.int32),
        ],
        compiler_params=_sc_params(),
    )
    def k(dst_hbm, out_hbm, hist_v, idx_v):
        c = lax.axis_index("c")
        s = lax.axis_index("s")
        wid = s * NC + c
        zero16 = jnp.zeros((16,), f32)
        one16 = jnp.full((16,), 1.0, f32)

        @pl.loop(0, N // 16)
        def _(i):
            hist_v[pl.ds(i * 16, 16)] = zero16

        per = E_ROWS // NW
        rem = E_ROWS % NW
        lo = wid * per + jnp.minimum(wid, rem)
        cnt = per + (wid < rem).astype(jnp.int32)

        @pl.loop(lo, lo + cnt)
        def _(r):
            pltpu.sync_copy(dst_hbm.at[r], idx_v)
            for j in range(CH // 16):
                idx = idx_v[pl.ds(j * 16, 16)]
                plsc.addupdate_scatter(hist_v, [idx], one16)

        pltpu.sync_copy(hist_v, out_hbm.at[wid])

    return k(dst2)


def _sc_agg(g, src2, dst2, feature_split):
    @functools.partial(
        pl.kernel,
        out_type=jax.ShapeDtypeStruct((NC, N, DH), f32),
        mesh=_mesh(),
        scratch_types=[
            pltpu.VMEM_SHARED((N, DH), f32),
            pltpu.VMEM((CH,), jnp.int32),
            pltpu.VMEM((CH,), jnp.int32),
            pltpu.VMEM((CH, DH), f32),
            pltpu.SemaphoreType.DMA,
        ],
        compiler_params=_sc_params(),
    )
    def k(g_hbm, src_hbm, dst_hbm, out_hbm, acc_sh, src_v, dst_v, rows_v, sem):
        c = lax.axis_index("c")
        s = lax.axis_index("s")

        if feature_split:
            @pl.loop(0, NRB)
            def _(b):
                @pl.when(lax.rem(b, NS) == s)
                def _():
                    pltpu.sync_copy(g_hbm.at[c].at[pl.ds(b * RB, RB)],
                                    rows_v.at[pl.ds(0, RB)])
                    pltpu.sync_copy(rows_v.at[pl.ds(0, RB)],
                                    acc_sh.at[pl.ds(b * RB, RB)])
        else:
            zero16 = jnp.zeros((16,), f32)

            @pl.when(c == 1)
            def _():
                @pl.loop(0, RB)
                def _(r):
                    for j in range(DH // 16):
                        rows_v[r, pl.ds(j * 16, 16)] = zero16

            @pl.loop(0, NRB)
            def _(b):
                @pl.when(lax.rem(b, NS) == s)
                def _():
                    @pl.when(c == 0)
                    def _():
                        pltpu.sync_copy(g_hbm.at[pl.ds(b * RB, RB)],
                                        rows_v.at[pl.ds(0, RB)])
                    pltpu.sync_copy(rows_v.at[pl.ds(0, RB)],
                                    acc_sh.at[pl.ds(b * RB, RB)])

        plsc.subcore_barrier()

        if feature_split:
            per = E_ROWS // NS
            rem = E_ROWS % NS
            lo = s * per + jnp.minimum(s, rem)
            cnt = per + (s < rem).astype(jnp.int32)
        else:
            half = E_ROWS // NC
            per = half // NS
            rem = half % NS
            lo = c * half + s * per + jnp.minimum(s, rem)
            cnt = per + (s < rem).astype(jnp.int32)

        gsrc = g_hbm.at[c] if feature_split else g_hbm

        @pl.loop(lo, lo + cnt)
        def _(r):
            pltpu.sync_copy(src_hbm.at[r], src_v)
            pltpu.sync_copy(dst_hbm.at[r], dst_v)
            pltpu.async_copy(gsrc.at[src_v], rows_v, sem).wait()
            pltpu.sync_copy(rows_v, acc_sh.at[dst_v], add=True)

        plsc.subcore_barrier()

        @pl.loop(0, NRB)
        def _(b):
            @pl.when(lax.rem(b, NS) == s)
            def _():
                pltpu.sync_copy(acc_sh.at[pl.ds(b * RB, RB)],
                                rows_v.at[pl.ds(0, RB)])
                pltpu.sync_copy(rows_v.at[pl.ds(0, RB)],
                                out_hbm.at[c].at[pl.ds(b * RB, RB)])

    return k(g, src2, dst2)


def _tc1(x, W1, histT):
    def body(x_ref, w_ref, h_ref, g_ref, dis_ref):
        cnt = jnp.sum(h_ref[...], axis=1, keepdims=True)
        dis = lax.rsqrt(cnt + 1.0)
        h1 = jnp.dot(x_ref[...], w_ref[...], preferred_element_type=f32)
        gg = h1 * dis
        g_ref[0] = gg[:, :DH]
        g_ref[1] = gg[:, DH:]
        dis_ref[...] = dis

    return pl.pallas_call(
        body,
        grid=(N // BR,),
        in_specs=[
            pl.BlockSpec((BR, D_IN), lambda i: (i, 0)),
            pl.BlockSpec((D_IN, D_HID), lambda i: (0, 0)),
            pl.BlockSpec((BR, NW), lambda i: (i, 0)),
        ],
        out_specs=[
            pl.BlockSpec((2, BR, DH), lambda i: (0, i, 0)),
            pl.BlockSpec((BR, 1), lambda i: (i, 0)),
        ],
        out_shape=[
            jax.ShapeDtypeStruct((NC, N, DH), f32),
            jax.ShapeDtypeStruct((N, 1), f32),
        ],
    )(x, W1, histT)


def _tc2(agg1, dis, b1, W2):
    def body(a_ref, dis_ref, b_ref, w_ref, o_ref):
        a = jnp.concatenate([a_ref[0], a_ref[1]], axis=1)
        d = dis_ref[...]
        z = jnp.maximum(a * d + b_ref[...], 0.0)
        h2 = jnp.dot(z, w_ref[...], preferred_element_type=f32)
        o_ref[...] = h2 * d

    return pl.pallas_call(
        body,
        grid=(N // BR,),
        in_specs=[
            pl.BlockSpec((2, BR, DH), lambda i: (0, i, 0)),
            pl.BlockSpec((BR, 1), lambda i: (i, 0)),
            pl.BlockSpec((1, D_HID), lambda i: (0, 0)),
            pl.BlockSpec((D_HID, D_OUT), lambda i: (0, 0)),
        ],
        out_specs=pl.BlockSpec((BR, D_OUT), lambda i: (i, 0)),
        out_shape=jax.ShapeDtypeStruct((N, D_OUT), f32),
    )(agg1, dis, b1, W2)


def _tc3(agg2, dis, b2):
    def body(a_ref, dis_ref, b_ref, o_ref):
        a = a_ref[0] + a_ref[1]
        o_ref[...] = jnp.maximum(a * dis_ref[...] + b_ref[...], 0.0)

    return pl.pallas_call(
        body,
        grid=(N // BR,),
        in_specs=[
            pl.BlockSpec((2, BR, DH), lambda i: (0, i, 0)),
            pl.BlockSpec((BR, 1), lambda i: (i, 0)),
            pl.BlockSpec((1, D_OUT), lambda i: (0, 0)),
        ],
        out_specs=pl.BlockSpec((BR, D_OUT), lambda i: (i, 0)),
        out_shape=jax.ShapeDtypeStruct((N, D_OUT), f32),
    )(agg2, dis, b2)


def kernel(x, edge_index, W1, b1, W2, b2):
    src2 = edge_index[0].reshape(E_ROWS, CH)
    dst2 = edge_index[1].reshape(E_ROWS, CH)

    hists = _sc_hist(dst2)
    histT = jnp.transpose(hists)

    g1, dis = _tc1(x, W1, histT)
    agg1 = _sc_agg(g1, src2, dst2, feature_split=True)
    g2 = _tc2(agg1, dis, b1.reshape(1, D_HID), W2)
    agg2 = _sc_agg(g2, src2, dst2, feature_split=False)
    out = _tc3(agg2, dis, b2.reshape(1, D_OUT))
    return out
